# Initial kernel scaffold; baseline (speedup 1.0000x reference)
import jax
import jax.numpy as jnp
from jax.experimental import pallas as pl


def kernel(x, edge_index, W1, b1, W2, b2):
    raise NotImplementedError("write your pallas kernel here")


# speedup vs baseline: 6.7027x; 6.7027x over previous
import functools

import jax
import jax.numpy as jnp
from jax import lax
from jax.experimental import pallas as pl
from jax.experimental.pallas import tpu as pltpu
from jax.experimental.pallas import tpu_sc as plsc

N = 10000
E = 320000
D = 128
H = 128
C = 40

W1EXT = 144
W2EXT = 48

NC = 2
NS = 16
NW = NC * NS
EPT = E // NW
K = 128
G_FULL = EPT // K
K_TAIL = EPT - G_FULL * K
N_PAD = 10240
ROWS_PER_TILE = N_PAD // NS

BLK = 1000


def _make_sc_aggregate(width):
    mesh = plsc.VectorSubcoreMesh(
        core_axis_name="c", subcore_axis_name="s", num_cores=NC, num_subcores=NS
    )

    @functools.partial(
        pl.kernel,
        mesh=mesh,
        compiler_params=pltpu.CompilerParams(use_tc_tiling_on_sc=False),
        out_type=jax.ShapeDtypeStruct((NC, N_PAD, width), jnp.float32),
        scratch_types=[
            pltpu.VMEM((K,), jnp.int32),
            pltpu.VMEM((K,), jnp.int32),
            pltpu.VMEM((K, width), jnp.float32),
            pltpu.VMEM((K_TAIL,), jnp.int32),
            pltpu.VMEM((K_TAIL,), jnp.int32),
            pltpu.VMEM((K_TAIL, width), jnp.float32),
            pltpu.VMEM_SHARED((N_PAD, width), jnp.float32),
            pltpu.SemaphoreType.DMA,
        ],
    )
    def agg(h_hbm, src_hbm, dst_hbm, zeros_hbm, out_hbm,
            idx_s, idx_d, rows, idx_st, idx_dt, rows_t, acc, sem):
        c = lax.axis_index("c")
        s = lax.axis_index("s")
        wid = c * NS + s
        r0 = s * ROWS_PER_TILE

        pltpu.sync_copy(zeros_hbm.at[pl.ds(r0, ROWS_PER_TILE)],
                        acc.at[pl.ds(r0, ROWS_PER_TILE)])
        plsc.subcore_barrier()

        base = wid * EPT

        @pl.loop(0, G_FULL)
        def _(g):
            off = base + g * K
            pltpu.sync_copy(src_hbm.at[pl.ds(off, K)], idx_s)
            pltpu.sync_copy(dst_hbm.at[pl.ds(off, K)], idx_d)
            pltpu.async_copy(h_hbm.at[idx_s], rows, sem).wait()
            pltpu.sync_copy(rows, acc.at[idx_d], add=True)

        off = base + G_FULL * K
        pltpu.sync_copy(src_hbm.at[pl.ds(off, K_TAIL)], idx_st)
        pltpu.sync_copy(dst_hbm.at[pl.ds(off, K_TAIL)], idx_dt)
        pltpu.async_copy(h_hbm.at[idx_st], rows_t, sem).wait()
        pltpu.sync_copy(rows_t, acc.at[idx_dt], add=True)

        plsc.subcore_barrier()
        pltpu.sync_copy(acc.at[pl.ds(r0, ROWS_PER_TILE)],
                        out_hbm.at[c, pl.ds(r0, ROWS_PER_TILE)])

    return agg


_agg1 = _make_sc_aggregate(W1EXT)
_agg2 = _make_sc_aggregate(W2EXT)


def _mm1_body(x_ref, w_ref, o_ref):
    h = jnp.dot(x_ref[...], w_ref[...],
                preferred_element_type=jnp.float32,
                precision=lax.Precision.HIGHEST)
    o_ref[:, :D] = h
    col = lax.broadcasted_iota(jnp.int32, (BLK, W1EXT - D), 1)
    o_ref[:, D:] = jnp.where(col == 0, 1.0, 0.0)


def _mm1(x, w1):
    return pl.pallas_call(
        _mm1_body,
        grid=(N // BLK,),
        in_specs=[
            pl.BlockSpec((BLK, D), lambda i: (i, 0)),
            pl.BlockSpec((D, H), lambda i: (0, 0)),
        ],
        out_specs=pl.BlockSpec((BLK, W1EXT), lambda i: (i, 0)),
        out_shape=jax.ShapeDtypeStruct((N, W1EXT), jnp.float32),
    )(x, w1)


def _fin1_body(a_ref, b1_ref, w2_ref, h2_ref, rdeg_ref):
    su = a_ref[0] + a_ref[1]
    deg = jnp.maximum(su[:, D:D + 1], 1.0)
    rdeg = 1.0 / deg
    h1 = jnp.maximum(su[:, :D] * rdeg + b1_ref[...], 0.0)
    h2_ref[...] = jnp.dot(h1, w2_ref[...],
                          preferred_element_type=jnp.float32,
                          precision=lax.Precision.HIGHEST)
    rdeg_ref[...] = rdeg


def _fin1(acc, b1, w2p):
    return pl.pallas_call(
        _fin1_body,
        grid=(N // BLK,),
        in_specs=[
            pl.BlockSpec((NC, BLK, W1EXT), lambda i: (0, i, 0)),
            pl.BlockSpec((1, H), lambda i: (0, 0)),
            pl.BlockSpec((H, W2EXT), lambda i: (0, 0)),
        ],
        out_specs=[
            pl.BlockSpec((BLK, W2EXT), lambda i: (i, 0)),
            pl.BlockSpec((BLK, 1), lambda i: (i, 0)),
        ],
        out_shape=[
            jax.ShapeDtypeStruct((N, W2EXT), jnp.float32),
            jax.ShapeDtypeStruct((N, 1), jnp.float32),
        ],
    )(acc, b1, w2p)


def _fin2_body(a_ref, rdeg_ref, b2_ref, o_ref):
    o_ref[...] = (a_ref[0] + a_ref[1]) * rdeg_ref[...] + b2_ref[...]


def _fin2(acc, rdeg, b2p):
    return pl.pallas_call(
        _fin2_body,
        grid=(N // BLK,),
        in_specs=[
            pl.BlockSpec((NC, BLK, W2EXT), lambda i: (0, i, 0)),
            pl.BlockSpec((BLK, 1), lambda i: (i, 0)),
            pl.BlockSpec((1, W2EXT), lambda i: (0, 0)),
        ],
        out_specs=pl.BlockSpec((BLK, W2EXT), lambda i: (i, 0)),
        out_shape=jax.ShapeDtypeStruct((N, W2EXT), jnp.float32),
    )(acc, rdeg, b2p)


def kernel(x, edge_index, W1, b1, W2, b2):
    src = edge_index[0]
    dst = edge_index[1]

    hext = _mm1(x, W1)
    zeros1 = jnp.zeros((N_PAD, W1EXT), jnp.float32)
    acc1 = _agg1(hext, src, dst, zeros1)

    w2p = jnp.pad(W2, ((0, 0), (0, W2EXT - C)))
    h2, rdeg = _fin1(acc1, b1.reshape(1, H), w2p)

    zeros2 = jnp.zeros((N_PAD, W2EXT), jnp.float32)
    acc2 = _agg2(h2, src, dst, zeros2)

    b2p = jnp.pad(b2, (0, W2EXT - C)).reshape(1, W2EXT)
    out = _fin2(acc2, rdeg, b2p)
    return out[:, :C]
